# baseline (speedup 1.0000x reference)
import functools

import jax
import jax.numpy as jnp
from jax import lax
from jax.experimental import pallas as pl
from jax.experimental.pallas import tpu as pltpu
from jax.experimental.pallas import tpu_sc as plsc

EXT = 16
NSEG = 8
FEAT = NSEG * EXT
COEFFS = (1.0, 0.5, 0.25, 0.125, 2.0, 1.5, 0.75, 0.375)

NC = 2
NS = 16
NW = NC * NS


def _sc_partials(x0, x1, src, dst2, zinit):
    n, feat = x0.shape
    e = x1.shape[0] // EXT
    per_w = e // NW
    ch = 40
    nchunk = per_w // ch
    nbuf = 5
    ngroup = nchunk // nbuf
    rpt = (n // NS) // 8 * 8
    tail = n - rpt * NS

    mesh = plsc.VectorSubcoreMesh(
        core_axis_name="c", subcore_axis_name="s", num_cores=NC, num_subcores=NS
    )

    @functools.partial(
        pl.kernel,
        mesh=mesh,
        out_type=jax.ShapeDtypeStruct((NC, n, feat), jnp.float32),
        scratch_types=[
            pltpu.VMEM((nbuf * ch,), jnp.int32),
            pltpu.VMEM((nbuf, ch), jnp.int32),
            pltpu.VMEM((nbuf * ch * EXT,), jnp.float32),
            pltpu.VMEM((nbuf, ch, FEAT), jnp.float32),
            pltpu.VMEM_SHARED((n, feat), jnp.float32),
            pltpu.SemaphoreType.DMA((nbuf,)),
            pltpu.SemaphoreType.DMA((nbuf,)),
            pltpu.SemaphoreType.DMA((nbuf,)),
            pltpu.SemaphoreType.DMA((nbuf,)),
            pltpu.SemaphoreType.DMA((nbuf,)),
        ],
    )
    def k(x0_hbm, x1_hbm, src_hbm, dst_hbm, z_hbm, out_hbm,
          srcr, dstr, x1c, rows, acc_sh, gsem, ssem, xsem, isem, dsem):
        c = lax.axis_index("c")
        s = lax.axis_index("s")
        wid = s * NC + c

        pltpu.sync_copy(z_hbm.at[pl.ds(s * rpt, rpt)],
                        acc_sh.at[pl.ds(s * rpt, rpt)])

        @pl.when(s == 0)
        def _():
            pltpu.sync_copy(z_hbm.at[pl.ds(NS * rpt, tail)],
                            acc_sh.at[pl.ds(NS * rpt, tail)])

        plsc.subcore_barrier()

        wbase = wid * per_w
        wrow = wid * nchunk

        def fetch_idx(jj, b):
            pltpu.async_copy(src_hbm.at[pl.ds(wbase + jj * ch, ch)],
                             srcr.at[pl.ds(b * ch, ch)], isem.at[b])
            pltpu.async_copy(dst_hbm.at[pl.ds(wrow + jj, 1)],
                             dstr.at[pl.ds(b, 1)], dsem.at[b])
            pltpu.async_copy(
                x1_hbm.at[pl.ds((wbase + jj * ch) * EXT, ch * EXT)],
                x1c.at[pl.ds(b * ch * EXT, ch * EXT)], xsem.at[b])

        def fetch_rows(jj, b):
            pltpu.make_async_copy(src_hbm.at[pl.ds(wbase + jj * ch, ch)],
                                  srcr.at[pl.ds(b * ch, ch)], isem.at[b]).wait()
            pltpu.async_copy(x0_hbm.at[srcr.at[pl.ds(b * ch, ch)]],
                             rows.at[b], gsem.at[b])

        for b in range(nbuf - 1):
            fetch_idx(b, b)
        for b in range(nbuf - 1):
            fetch_rows(b, b)

        def group(g, carry):
            for b in range(nbuf):
                jj = g * nbuf + b
                pltpu.make_async_copy(
                    x0_hbm.at[srcr.at[pl.ds(b * ch, ch)]],
                    rows.at[b], gsem.at[b]).wait()
                pltpu.make_async_copy(
                    x1_hbm.at[pl.ds((wbase + jj * ch) * EXT, ch * EXT)],
                    x1c.at[pl.ds(b * ch * EXT, ch * EXT)], xsem.at[b]).wait()
                pltpu.make_async_copy(
                    dst_hbm.at[pl.ds(wrow + jj, 1)],
                    dstr.at[pl.ds(b, 1)], dsem.at[b]).wait()

                @plsc.parallel_loop(0, ch, 1, unroll=16)
                def _(i, b=b):
                    x1e = x1c[pl.ds((b * ch + i) * EXT, EXT)]
                    for seg in range(NSEG):
                        sl = pl.ds(seg * EXT, EXT)
                        rows[b, i, sl] = rows[b, i, sl] * (x1e * COEFFS[seg])
                pltpu.async_copy(rows.at[b], acc_sh.at[dstr.at[b]],
                                 ssem.at[b], add=True)

                nb = (b + nbuf - 1) % nbuf
                jn = jj + nbuf - 1

                @pl.when(jn < nchunk)
                def _(jj=jj, jn=jn, nb=nb):
                    @pl.when(jj >= 1)
                    def _():
                        pltpu.make_async_copy(
                            rows.at[nb], acc_sh.at[dstr.at[nb]],
                            ssem.at[nb]).wait()
                    fetch_idx(jn, nb)
                    fetch_rows(jn, nb)
            return carry

        lax.fori_loop(0, ngroup, group, 0)

        for b in range(nbuf):
            pltpu.make_async_copy(
                rows.at[b], acc_sh.at[dstr.at[b]], ssem.at[b]).wait()

        plsc.subcore_barrier()
        pltpu.sync_copy(acc_sh.at[pl.ds(s * rpt, rpt)],
                        out_hbm.at[c, pl.ds(s * rpt, rpt)])

        @pl.when(s == 0)
        def _():
            pltpu.sync_copy(acc_sh.at[pl.ds(NS * rpt, tail)],
                            out_hbm.at[c, pl.ds(NS * rpt, tail)])

    return k(x0, x1, src, dst2, zinit)


def _tc_sum(partials):
    nc, n, feat = partials.shape
    br = 1000

    def body(p_ref, o_ref):
        o_ref[...] = p_ref[0] + p_ref[1]

    return pl.pallas_call(
        body,
        grid=(n // br,),
        in_specs=[pl.BlockSpec((nc, br, feat), lambda i: (0, i, 0))],
        out_specs=pl.BlockSpec((br, feat), lambda i: (i, 0)),
        out_shape=jax.ShapeDtypeStruct((n, feat), jnp.float32),
    )(partials)


def kernel(x0, x1, src, dst, n_out):
    del n_out
    e = x1.shape[0]
    ch = 40
    dst2 = dst.reshape(e // ch, ch)
    x1f = x1.reshape(-1)
    zinit = jnp.zeros(x0.shape, jnp.float32)
    partials = _sc_partials(x0, x1f, src, dst2, zinit)
    return _tc_sum(partials)


# speedup vs baseline: 1.1056x; 1.1056x over previous
import functools

import jax
import jax.numpy as jnp
from jax import lax
from jax.experimental import pallas as pl
from jax.experimental.pallas import tpu as pltpu
from jax.experimental.pallas import tpu_sc as plsc

EXT = 16
NSEG = 8
FEAT = NSEG * EXT
COEFFS = (1.0, 0.5, 0.25, 0.125, 2.0, 1.5, 0.75, 0.375)

NC = 2
NS = 16
NW = NC * NS


def _sc_partials(x0, x1, src, dst2, zinit):
    n, feat = x0.shape
    e = x1.shape[0]
    per_w = e // NW
    ch = 40
    nchunk = per_w // ch
    nbuf = 5
    ngroup = nchunk // nbuf
    rpt = (n // NS) // 8 * 8
    tail = n - rpt * NS

    mesh = plsc.VectorSubcoreMesh(
        core_axis_name="c", subcore_axis_name="s", num_cores=NC, num_subcores=NS
    )

    @functools.partial(
        pl.kernel,
        mesh=mesh,
        out_type=jax.ShapeDtypeStruct((NC, n, feat), jnp.float32),
        scratch_types=[
            pltpu.VMEM((nbuf * ch,), jnp.int32),
            pltpu.VMEM((nbuf, ch), jnp.int32),
            pltpu.VMEM((nbuf - 1, ch, EXT), jnp.float32),
            pltpu.VMEM((nbuf, ch, FEAT), jnp.float32),
            pltpu.VMEM_SHARED((n, feat), jnp.float32),
            pltpu.SemaphoreType.DMA((nbuf,)),
            pltpu.SemaphoreType.DMA((nbuf,)),
            pltpu.SemaphoreType.DMA((nbuf - 1,)),
            pltpu.SemaphoreType.DMA((nbuf,)),
            pltpu.SemaphoreType.DMA((nbuf,)),
        ],
    )
    def k(x0_hbm, x1_hbm, src_hbm, dst_hbm, z_hbm, out_hbm,
          srcr, dstr, x1c, rows, acc_sh, gsem, ssem, xsem, isem, dsem):
        c = lax.axis_index("c")
        s = lax.axis_index("s")
        wid = s * NC + c

        pltpu.sync_copy(z_hbm.at[pl.ds(s * rpt, rpt)],
                        acc_sh.at[pl.ds(s * rpt, rpt)])

        @pl.when(s == 0)
        def _():
            pltpu.sync_copy(z_hbm.at[pl.ds(NS * rpt, tail)],
                            acc_sh.at[pl.ds(NS * rpt, tail)])

        plsc.subcore_barrier()

        wbase = wid * per_w
        wrow = wid * nchunk

        def fetch_idx(jj, b):
            pltpu.async_copy(src_hbm.at[pl.ds(wbase + jj * ch, ch)],
                             srcr.at[pl.ds(b * ch, ch)], isem.at[b])
            pltpu.async_copy(dst_hbm.at[pl.ds(wrow + jj, 1)],
                             dstr.at[pl.ds(b, 1)], dsem.at[b])
            x4 = lax.rem(jj, nbuf - 1) if not isinstance(jj, int) else jj % (nbuf - 1)
            pltpu.async_copy(x1_hbm.at[pl.ds(wbase + jj * ch, ch)],
                             x1c.at[x4], xsem.at[x4])

        def fetch_rows(jj, b):
            pltpu.make_async_copy(src_hbm.at[pl.ds(wbase + jj * ch, ch)],
                                  srcr.at[pl.ds(b * ch, ch)], isem.at[b]).wait()
            pltpu.async_copy(x0_hbm.at[srcr.at[pl.ds(b * ch, ch)]],
                             rows.at[b], gsem.at[b])

        for b in range(nbuf - 1):
            fetch_idx(b, b)
        for b in range(nbuf - 1):
            fetch_rows(b, b)

        def group(g, carry):
            for b in range(nbuf):
                jj = g * nbuf + b
                x4 = lax.rem(jj, nbuf - 1)
                pltpu.make_async_copy(
                    x0_hbm.at[srcr.at[pl.ds(b * ch, ch)]],
                    rows.at[b], gsem.at[b]).wait()
                pltpu.make_async_copy(
                    x1_hbm.at[pl.ds(wbase + jj * ch, ch)],
                    x1c.at[x4], xsem.at[x4]).wait()
                pltpu.make_async_copy(
                    dst_hbm.at[pl.ds(wrow + jj, 1)],
                    dstr.at[pl.ds(b, 1)], dsem.at[b]).wait()

                @plsc.parallel_loop(0, ch, 1, unroll=8)
                def _(i, b=b, x4=x4):
                    x1e = x1c[x4, i, :]
                    for seg in range(NSEG):
                        sl = pl.ds(seg * EXT, EXT)
                        rows[b, i, sl] = rows[b, i, sl] * (x1e * COEFFS[seg])
                pltpu.async_copy(rows.at[b], acc_sh.at[dstr.at[b]],
                                 ssem.at[b], add=True)

                nb = (b + nbuf - 1) % nbuf
                jn = jj + nbuf - 1

                @pl.when(jn < nchunk)
                def _(jj=jj, jn=jn, nb=nb):
                    @pl.when(jj >= 1)
                    def _():
                        pltpu.make_async_copy(
                            rows.at[nb], acc_sh.at[dstr.at[nb]],
                            ssem.at[nb]).wait()
                    fetch_idx(jn, nb)
                    fetch_rows(jn, nb)
            return carry

        lax.fori_loop(0, ngroup, group, 0)

        for b in range(nbuf):
            pltpu.make_async_copy(
                rows.at[b], acc_sh.at[dstr.at[b]], ssem.at[b]).wait()

        plsc.subcore_barrier()
        pltpu.sync_copy(acc_sh.at[pl.ds(s * rpt, rpt)],
                        out_hbm.at[c, pl.ds(s * rpt, rpt)])

        @pl.when(s == 0)
        def _():
            pltpu.sync_copy(acc_sh.at[pl.ds(NS * rpt, tail)],
                            out_hbm.at[c, pl.ds(NS * rpt, tail)])

    return k(x0, x1, src, dst2, zinit)


def _tc_sum(partials):
    nc, n, feat = partials.shape
    br = 1000

    def body(p_ref, o_ref):
        o_ref[...] = p_ref[0] + p_ref[1]

    return pl.pallas_call(
        body,
        grid=(n // br,),
        in_specs=[pl.BlockSpec((nc, br, feat), lambda i: (0, i, 0))],
        out_specs=pl.BlockSpec((br, feat), lambda i: (i, 0)),
        out_shape=jax.ShapeDtypeStruct((n, feat), jnp.float32),
    )(partials)


def kernel(x0, x1, src, dst, n_out):
    del n_out
    e = x1.shape[0]
    ch = 40
    dst2 = dst.reshape(e // ch, ch)
    zinit = jnp.zeros(x0.shape, jnp.float32)
    partials = _sc_partials(x0, x1, src, dst2, zinit)
    return _tc_sum(partials)


# speedup vs baseline: 1.2504x; 1.1309x over previous
import functools

import jax
import jax.numpy as jnp
from jax import lax
from jax.experimental import pallas as pl
from jax.experimental.pallas import tpu as pltpu
from jax.experimental.pallas import tpu_sc as plsc

EXT = 16
NSEG = 8
FEAT = NSEG * EXT
COEFFS = (1.0, 0.5, 0.25, 0.125, 2.0, 1.5, 0.75, 0.375)

NC = 2
NS = 16
NW = NC * NS
CH = 40
NBUF = 5
NIG = 3


def _sc_partials(x0, x1, src, dst4, zinit):
    n, feat = x0.shape
    e = x1.shape[0]
    per_w = e // NW
    ch = CH
    nchunk = per_w // ch
    nbuf = NBUF
    ngroup = nchunk // nbuf
    rpt = (n // NS) // 8 * 8
    tail = n - rpt * NS
    gch = nbuf * ch

    mesh = plsc.VectorSubcoreMesh(
        core_axis_name="c", subcore_axis_name="s", num_cores=NC, num_subcores=NS
    )

    @functools.partial(
        pl.kernel,
        mesh=mesh,
        out_type=jax.ShapeDtypeStruct((NC, n, feat), jnp.float32),
        scratch_types=[
            pltpu.VMEM((NIG * gch,), jnp.int32),
            pltpu.VMEM((NIG * nbuf, ch), jnp.int32),
            pltpu.VMEM((nbuf - 1, ch, EXT), jnp.float32),
            pltpu.VMEM((nbuf, ch, FEAT), jnp.float32),
            pltpu.VMEM_SHARED((n, feat), jnp.float32),
            pltpu.SemaphoreType.DMA((nbuf,)),
            pltpu.SemaphoreType.DMA((nbuf,)),
            pltpu.SemaphoreType.DMA((nbuf - 1,)),
            pltpu.SemaphoreType.DMA((NIG,)),
            pltpu.SemaphoreType.DMA((NIG,)),
        ],
    )
    def k(x0_hbm, x1_hbm, src_hbm, dst_hbm, z_hbm, out_hbm,
          srcr, dstr, x1c, rows, acc_sh, gsem, ssem, xsem, isem, dsem):
        c = lax.axis_index("c")
        s = lax.axis_index("s")
        wid = s * NC + c

        pltpu.sync_copy(z_hbm.at[pl.ds(s * rpt, rpt)],
                        acc_sh.at[pl.ds(s * rpt, rpt)])

        @pl.when(s == 0)
        def _():
            pltpu.sync_copy(z_hbm.at[pl.ds(NS * rpt, tail)],
                            acc_sh.at[pl.ds(NS * rpt, tail)])

        plsc.subcore_barrier()

        wbase = wid * per_w

        def load_group(g, sl):
            pltpu.async_copy(src_hbm.at[pl.ds(wbase + g * gch, gch)],
                             srcr.at[pl.ds(sl * gch, gch)], isem.at[sl])
            pltpu.async_copy(dst_hbm.at[wid, g],
                             dstr.at[pl.ds(sl * nbuf, nbuf)], dsem.at[sl])

        def wait_group_src(g, sl):
            pltpu.make_async_copy(src_hbm.at[pl.ds(wbase + g * gch, gch)],
                                  srcr.at[pl.ds(sl * gch, gch)],
                                  isem.at[sl]).wait()

        def wait_group_dst(g, sl):
            pltpu.make_async_copy(dst_hbm.at[wid, g],
                                  dstr.at[pl.ds(sl * nbuf, nbuf)],
                                  dsem.at[sl]).wait()

        def fetch_x1(jj, b):
            x4 = lax.rem(jj, nbuf - 1) if not isinstance(jj, int) \
                else jj % (nbuf - 1)
            pltpu.async_copy(x1_hbm.at[pl.ds(wbase + jj * ch, ch)],
                             x1c.at[x4], xsem.at[x4])

        def fetch_rows(jj, b, sl_b):
            pltpu.async_copy(x0_hbm.at[srcr.at[pl.ds(sl_b * ch, ch)]],
                             rows.at[b], gsem.at[b])

        for gg in range(2):
            load_group(gg, gg)
        wait_group_src(0, 0)
        for b in range(nbuf - 1):
            fetch_x1(b, b)
            fetch_rows(b, b, b)

        def group(g, carry):
            sl = lax.rem(g, NIG)
            for b in range(nbuf):
                jj = g * nbuf + b
                x4 = lax.rem(jj, nbuf - 1)

                if b == 0:
                    wait_group_dst(g, sl)

                pltpu.make_async_copy(
                    x0_hbm.at[srcr.at[pl.ds((sl * nbuf + b) * ch, ch)]],
                    rows.at[b], gsem.at[b]).wait()
                pltpu.make_async_copy(
                    x1_hbm.at[pl.ds(wbase + jj * ch, ch)],
                    x1c.at[x4], xsem.at[x4]).wait()

                @plsc.parallel_loop(0, ch, 1, unroll=8)
                def _(i, b=b, x4=x4):
                    x1e = x1c[x4, i, :]
                    for seg in range(NSEG):
                        slc = pl.ds(seg * EXT, EXT)
                        rows[b, i, slc] = rows[b, i, slc] * (x1e * COEFFS[seg])

                pltpu.async_copy(rows.at[b], acc_sh.at[dstr.at[sl * nbuf + b]],
                                 ssem.at[b], add=True)

                nb = (b + nbuf - 1) % nbuf
                jn = jj + nbuf - 1

                @pl.when(jn < nchunk)
                def _(g=g, jj=jj, jn=jn, nb=nb, b=b, sl=sl):
                    @pl.when(jj >= 1)
                    def _():
                        gp = (jj - 1) // nbuf
                        pltpu.make_async_copy(
                            rows.at[nb],
                            acc_sh.at[dstr.at[lax.rem(gp, NIG) * nbuf + nb]],
                            ssem.at[nb]).wait()
                    if b == 1:
                        wait_group_src(g + 1, lax.rem(g + 1, NIG))
                    fetch_x1(jn, nb)
                    gn = jn // nbuf
                    fetch_rows(jn, nb,
                               lax.rem(gn, NIG) * nbuf + (jn - gn * nbuf))

                if b == 0:
                    @pl.when(g < ngroup - 2)
                    def _(g=g):
                        load_group(g + 2, lax.rem(g + 2, NIG))
            return carry

        lax.fori_loop(0, ngroup, group, 0)

        lsl = (ngroup - 1) % NIG
        for b in range(nbuf):
            pltpu.make_async_copy(
                rows.at[b], acc_sh.at[dstr.at[lsl * nbuf + b]],
                ssem.at[b]).wait()

        plsc.subcore_barrier()
        pltpu.sync_copy(acc_sh.at[pl.ds(s * rpt, rpt)],
                        out_hbm.at[c, pl.ds(s * rpt, rpt)])

        @pl.when(s == 0)
        def _():
            pltpu.sync_copy(acc_sh.at[pl.ds(NS * rpt, tail)],
                            out_hbm.at[c, pl.ds(NS * rpt, tail)])

    return k(x0, x1, src, dst4, zinit)


def _tc_sum(partials):
    nc, n, feat = partials.shape
    br = 1000

    def body(p_ref, o_ref):
        o_ref[...] = p_ref[0] + p_ref[1]

    return pl.pallas_call(
        body,
        grid=(n // br,),
        in_specs=[pl.BlockSpec((nc, br, feat), lambda i: (0, i, 0))],
        out_specs=pl.BlockSpec((br, feat), lambda i: (i, 0)),
        out_shape=jax.ShapeDtypeStruct((n, feat), jnp.float32),
    )(partials)


def kernel(x0, x1, src, dst, n_out):
    del n_out
    e = x1.shape[0]
    per_w = e // NW
    dst4 = dst.reshape(NW, per_w // (NBUF * CH), NBUF, CH)
    zinit = jnp.zeros(x0.shape, jnp.float32)
    partials = _sc_partials(x0, x1, src, dst4, zinit)
    return _tc_sum(partials)
